# baseline (device time: 55259 ns/iter reference)
import os

import jax
import jax.numpy as jnp
from jax import lax
from jax.experimental import pallas as pl
from jax.experimental.pallas import tpu as pltpu

N_DEV = 4
P = 2

try:
    _MODE = (
        open(os.path.join(os.path.dirname(__file__), "a2a_mode.txt")).read().strip()
    )
except OSError:
    _MODE = "full"
_OFFS = {"full": (1, 2, 3), "nodiag": (1, 3), "local": ()}[_MODE]


def kernel(x):
    m_per, n = x.shape
    n_per = n // N_DEV
    out_rows = m_per * N_DEV
    m_piece = m_per // P

    def body(x_ref, out_ref, xc_ref, xb_ref, rx_ref, load_sems, store_sems,
             send_sems, recv_sems):
        me = lax.axis_index("i")

        def slot(off, p):
            return off * P + p

        loads = {}
        for p in range(P):
            for off in _OFFS:
                peer = lax.rem(me + off, N_DEV)
                cp = pltpu.make_async_copy(
                    x_ref.at[pl.ds(p * m_piece, m_piece),
                             pl.ds(peer * n_per, n_per)],
                    xc_ref.at[slot(off, p)],
                    load_sems.at[slot(off, p)],
                )
                cp.start()
                loads[off, p] = cp
        for p in range(P):
            cp = pltpu.make_async_copy(
                x_ref.at[pl.ds(p * m_piece, m_piece),
                         pl.ds(me * n_per, n_per)],
                xc_ref.at[slot(0, p)],
                load_sems.at[slot(0, p)],
            )
            cp.start()
            loads[0, p] = cp

        barrier_sem = pltpu.get_barrier_semaphore()
        for off in range(1, N_DEV):
            peer = lax.rem(me + off, N_DEV)
            pl.semaphore_signal(
                barrier_sem, inc=1,
                device_id=(peer,), device_id_type=pl.DeviceIdType.MESH,
            )
        pl.semaphore_wait(barrier_sem, N_DEV - 1)

        sends = []
        for p in range(P):
            for off in _OFFS:
                peer = lax.rem(me + off, N_DEV)
                loads[off, p].wait()
                xb_ref[slot(off, p)] = xc_ref[slot(off, p)].astype(jnp.bfloat16)
                rdma = pltpu.make_async_remote_copy(
                    src_ref=xb_ref.at[slot(off, p)],
                    dst_ref=rx_ref.at[me, p],
                    send_sem=send_sems.at[slot(off - 1, p)],
                    recv_sem=recv_sems.at[me, p],
                    device_id=(peer,),
                    device_id_type=pl.DeviceIdType.MESH,
                )
                rdma.start()
                sends.append(rdma)

        stores = []
        for p in range(P):
            loads[0, p].wait()
            xb_ref[slot(0, p)] = xc_ref[slot(0, p)].astype(jnp.bfloat16)
            store = pltpu.make_async_copy(
                xb_ref.at[slot(0, p)],
                out_ref.at[pl.ds(me * m_per + p * m_piece, m_piece), :],
                store_sems.at[slot(0, p)],
            )
            store.start()
            stores.append(store)

        for p in range(P):
            for off in _OFFS:
                src = lax.rem(me - off + N_DEV, N_DEV)
                recv = pltpu.make_async_remote_copy(
                    src_ref=xb_ref.at[slot(0, p)],
                    dst_ref=rx_ref.at[src, p],
                    send_sem=send_sems.at[slot(off - 1, p)],
                    recv_sem=recv_sems.at[src, p],
                    device_id=(src,),
                    device_id_type=pl.DeviceIdType.MESH,
                )
                recv.wait_recv()
                store = pltpu.make_async_copy(
                    rx_ref.at[src, p],
                    out_ref.at[pl.ds(src * m_per + p * m_piece, m_piece), :],
                    store_sems.at[slot(off, p)],
                )
                store.start()
                stores.append(store)

        for store in stores:
            store.wait()
        for rdma in sends:
            rdma.wait_send()

    return pl.pallas_call(
        body,
        out_shape=jax.ShapeDtypeStruct((out_rows, n_per), jnp.bfloat16),
        in_specs=[pl.BlockSpec(memory_space=pl.ANY)],
        out_specs=pl.BlockSpec(memory_space=pl.ANY),
        scratch_shapes=[
            pltpu.VMEM((N_DEV * P, m_piece, n_per), jnp.float32),
            pltpu.VMEM((N_DEV * P, m_piece, n_per), jnp.bfloat16),
            pltpu.VMEM((N_DEV, P, m_piece, n_per), jnp.bfloat16),
            pltpu.SemaphoreType.DMA((N_DEV * P,)),
            pltpu.SemaphoreType.DMA((N_DEV * P,)),
            pltpu.SemaphoreType.DMA(((N_DEV - 1) * P,)),
            pltpu.SemaphoreType.DMA((N_DEV, P)),
        ],
        compiler_params=pltpu.CompilerParams(collective_id=0),
    )(x)


# device time: 54469 ns/iter; 1.0145x vs baseline; 1.0145x over previous
import os

import jax
import jax.numpy as jnp
from jax import lax
from jax.experimental import pallas as pl
from jax.experimental.pallas import tpu as pltpu

N_DEV = 4
P = 4

try:
    _MODE = (
        open(os.path.join(os.path.dirname(__file__), "a2a_mode.txt")).read().strip()
    )
except OSError:
    _MODE = "full"
_OFFS = {"full": (1, 2, 3), "nodiag": (1, 3), "local": ()}[_MODE]
_SEND_OFFS = {"full": (2, 1, 3), "nodiag": (1, 3), "local": ()}[_MODE]
_RECV_OFFS = {"full": (1, 3, 2), "nodiag": (1, 3), "local": ()}[_MODE]


def kernel(x):
    m_per, n = x.shape
    n_per = n // N_DEV
    out_rows = m_per * N_DEV
    m_piece = m_per // P

    def body(x_ref, out_ref, xc_ref, xb_ref, rx_ref, load_sems, store_sems,
             send_sems, recv_sems):
        me = lax.axis_index("i")

        def slot(off, p):
            return off * P + p

        loads = {}
        for p in range(P):
            for off in _SEND_OFFS:
                peer = lax.rem(me + off, N_DEV)
                cp = pltpu.make_async_copy(
                    x_ref.at[pl.ds(p * m_piece, m_piece),
                             pl.ds(peer * n_per, n_per)],
                    xc_ref.at[slot(off, p)],
                    load_sems.at[slot(off, p)],
                )
                cp.start()
                loads[off, p] = cp
        for p in range(P):
            cp = pltpu.make_async_copy(
                x_ref.at[pl.ds(p * m_piece, m_piece),
                         pl.ds(me * n_per, n_per)],
                xc_ref.at[slot(0, p)],
                load_sems.at[slot(0, p)],
            )
            cp.start()
            loads[0, p] = cp

        barrier_sem = pltpu.get_barrier_semaphore()
        for off in range(1, N_DEV):
            peer = lax.rem(me + off, N_DEV)
            pl.semaphore_signal(
                barrier_sem, inc=1,
                device_id=(peer,), device_id_type=pl.DeviceIdType.MESH,
            )
        pl.semaphore_wait(barrier_sem, N_DEV - 1)

        sends = []
        for p in range(P):
            for off in _SEND_OFFS:
                peer = lax.rem(me + off, N_DEV)
                loads[off, p].wait()
                xb_ref[slot(off, p)] = xc_ref[slot(off, p)].astype(jnp.bfloat16)
                rdma = pltpu.make_async_remote_copy(
                    src_ref=xb_ref.at[slot(off, p)],
                    dst_ref=rx_ref.at[me, p],
                    send_sem=send_sems.at[slot(off - 1, p)],
                    recv_sem=recv_sems.at[me, p],
                    device_id=(peer,),
                    device_id_type=pl.DeviceIdType.MESH,
                )
                rdma.start()
                sends.append(rdma)

        stores = []
        for p in range(P):
            loads[0, p].wait()
            xb_ref[slot(0, p)] = xc_ref[slot(0, p)].astype(jnp.bfloat16)
            store = pltpu.make_async_copy(
                xb_ref.at[slot(0, p)],
                out_ref.at[pl.ds(me * m_per + p * m_piece, m_piece), :],
                store_sems.at[slot(0, p)],
            )
            store.start()
            stores.append(store)

        for p in range(P):
            for off in _RECV_OFFS:
                src = lax.rem(me - off + N_DEV, N_DEV)
                recv = pltpu.make_async_remote_copy(
                    src_ref=xb_ref.at[slot(0, p)],
                    dst_ref=rx_ref.at[src, p],
                    send_sem=send_sems.at[slot(off - 1, p)],
                    recv_sem=recv_sems.at[src, p],
                    device_id=(src,),
                    device_id_type=pl.DeviceIdType.MESH,
                )
                recv.wait_recv()
                store = pltpu.make_async_copy(
                    rx_ref.at[src, p],
                    out_ref.at[pl.ds(src * m_per + p * m_piece, m_piece), :],
                    store_sems.at[slot(off, p)],
                )
                store.start()
                stores.append(store)

        for store in stores:
            store.wait()
        for rdma in sends:
            rdma.wait_send()

    return pl.pallas_call(
        body,
        out_shape=jax.ShapeDtypeStruct((out_rows, n_per), jnp.bfloat16),
        in_specs=[pl.BlockSpec(memory_space=pl.ANY)],
        out_specs=pl.BlockSpec(memory_space=pl.ANY),
        scratch_shapes=[
            pltpu.VMEM((N_DEV * P, m_piece, n_per), jnp.float32),
            pltpu.VMEM((N_DEV * P, m_piece, n_per), jnp.bfloat16),
            pltpu.VMEM((N_DEV, P, m_piece, n_per), jnp.bfloat16),
            pltpu.SemaphoreType.DMA((N_DEV * P,)),
            pltpu.SemaphoreType.DMA((N_DEV * P,)),
            pltpu.SemaphoreType.DMA(((N_DEV - 1) * P,)),
            pltpu.SemaphoreType.DMA((N_DEV, P)),
        ],
        compiler_params=pltpu.CompilerParams(collective_id=0),
    )(x)
